# baseline (speedup 1.0000x reference)
import functools

import jax
import jax.lax as lax
import jax.numpy as jnp
from jax.experimental import pallas as pl
from jax.experimental.pallas import tpu as pltpu


def _se_packed_kernel(x_ref, oh_ref, oht_ref, wdt_ref, bd_ref, wut_ref,
                      bu_ref, o_ref, *, nb, nrow, cpr):
    f32 = jnp.float32
    x = x_ref[...]
    l = x.shape[-1]
    rows = nb * nrow
    c_total = nrow * cpr
    x2 = x.reshape(rows, l)

    pooled_r = jnp.dot(x2, oh_ref[...],
                       preferred_element_type=f32)

    r_i = lax.broadcasted_iota(jnp.int32, (rows, c_total), 0)
    c_i = lax.broadcasted_iota(jnp.int32, (rows, c_total), 1)
    m = ((r_i % nrow) == (c_i // cpr)).astype(f32)

    lo_i = lax.broadcasted_iota(jnp.int32, (cpr, c_total), 0)
    ce_i = lax.broadcasted_iota(jnp.int32, (cpr, c_total), 1)
    e = ((ce_i % cpr) == lo_i).astype(f32)

    b_i = lax.broadcasted_iota(jnp.int32, (nb, rows), 0)
    rg_i = lax.broadcasted_iota(jnp.int32, (nb, rows), 1)
    gb = ((rg_i // nrow) == b_i).astype(f32)

    pooled = jnp.dot(gb, jnp.dot(pooled_r, e, preferred_element_type=f32) * m,
                     preferred_element_type=f32)

    h = jnp.dot(pooled, wdt_ref[...], preferred_element_type=f32)
    h = jnp.maximum(h + bd_ref[...], 0.0)
    s = jnp.dot(h, wut_ref[...], preferred_element_type=f32)
    s = jax.nn.sigmoid(s + bu_ref[...])

    rb_i = lax.broadcasted_iota(jnp.int32, (rows, nb), 0)
    bb_i = lax.broadcasted_iota(jnp.int32, (rows, nb), 1)
    gbt = ((rb_i // nrow) == bb_i).astype(f32)

    cc_i = lax.broadcasted_iota(jnp.int32, (c_total, cpr), 0)
    le_i = lax.broadcasted_iota(jnp.int32, (c_total, cpr), 1)
    et = ((cc_i % cpr) == le_i).astype(f32)

    s_rep = jnp.dot(gbt, s, preferred_element_type=f32)
    s_r = jnp.dot(s_rep * m, et, preferred_element_type=f32)

    slab = jnp.dot(s_r, oht_ref[...],
                   preferred_element_type=f32)

    o_ref[...] = (x2 * slab).reshape(nb, nrow, l)


def _pick(total, candidates):
    for c in candidates:
        if total % c == 0:
            return c
    return 1


def kernel(x_nchw, w_down, b_down, w_up, b_up):
    B, C, H, W = x_nchw.shape
    HW = H * W
    I = w_down.shape[0]
    dtype = x_nchw.dtype

    cpr = _pick(C, (16, 8, 4, 2))
    nrow = C // cpr
    L = cpr * HW
    nb = _pick(B, (8, 4, 2))

    x3 = x_nchw.reshape(B, nrow, L)

    seg = jnp.arange(L, dtype=jnp.int32) // HW
    ohT = (seg[None, :] == jnp.arange(cpr, dtype=jnp.int32)[:, None]).astype(
        jnp.float32)
    oh = ohT.T * jnp.float32(1.0 / HW)

    wdt = w_down.astype(jnp.float32).T
    bd2 = b_down.astype(jnp.float32).reshape(1, I)
    wut = w_up.astype(jnp.float32).T
    bu2 = b_up.astype(jnp.float32).reshape(1, C)

    block_bytes = nb * nrow * (((L + 127) // 128) * 128) * dtype.itemsize
    vmem = int(min(4 * block_bytes + (8 << 20), 96 << 20))

    body = functools.partial(_se_packed_kernel, nb=nb, nrow=nrow, cpr=cpr)
    out = pl.pallas_call(
        body,
        out_shape=jax.ShapeDtypeStruct((B, nrow, L), dtype),
        grid_spec=pltpu.PrefetchScalarGridSpec(
            num_scalar_prefetch=0,
            grid=(B // nb,),
            in_specs=[
                pl.BlockSpec((nb, nrow, L), lambda b: (b, 0, 0)),
                pl.BlockSpec((L, cpr), lambda b: (0, 0)),
                pl.BlockSpec((cpr, L), lambda b: (0, 0)),
                pl.BlockSpec((C, I), lambda b: (0, 0)),
                pl.BlockSpec((1, I), lambda b: (0, 0)),
                pl.BlockSpec((I, C), lambda b: (0, 0)),
                pl.BlockSpec((1, C), lambda b: (0, 0)),
            ],
            out_specs=pl.BlockSpec((nb, nrow, L), lambda b: (b, 0, 0)),
        ),
        compiler_params=pltpu.CompilerParams(
            dimension_semantics=("parallel",),
            vmem_limit_bytes=vmem,
        ),
    )(x3, oh, ohT, wdt, bd2, wut, bu2)

    return out.reshape(B, C, H, W)


# speedup vs baseline: 8.8310x; 8.8310x over previous
import functools

import jax
import jax.numpy as jnp
from jax.experimental import pallas as pl
from jax.experimental.pallas import tpu as pltpu


def _se_kernel(x_ref, wdt_ref, bd_ref, wut_ref, bu_ref, o_ref):
    x = x_ref[...]

    pooled = jnp.mean(x, axis=0, dtype=jnp.float32)

    h = jnp.dot(pooled, wdt_ref[...], preferred_element_type=jnp.float32)
    h = jnp.maximum(h + bd_ref[...], 0.0)
    s = jnp.dot(h, wut_ref[...], preferred_element_type=jnp.float32)
    s = jax.nn.sigmoid(s + bu_ref[...])

    o_ref[...] = x * s[None, :, :].astype(x.dtype)


def _pick(total, candidates):
    for c in candidates:
        if total % c == 0:
            return c
    return 1


def kernel(x_nchw, w_down, b_down, w_up, b_up):
    B, C, H, W = x_nchw.shape
    HW = H * W
    I = w_down.shape[0]
    dtype = x_nchw.dtype

    x_t = jnp.transpose(x_nchw, (2, 3, 0, 1)).reshape(HW, B, C)

    wdt = w_down.astype(jnp.float32).T
    bd2 = b_down.astype(jnp.float32).reshape(1, I)
    wut = w_up.astype(jnp.float32).T
    bu2 = b_up.astype(jnp.float32).reshape(1, C)

    nb = _pick(B, (16, 8, 4, 2))
    block_bytes = HW * nb * C * dtype.itemsize
    vmem = int(min(4 * block_bytes + (4 << 20), 100 << 20))

    out = pl.pallas_call(
        _se_kernel,
        out_shape=jax.ShapeDtypeStruct((HW, B, C), dtype),
        grid_spec=pltpu.PrefetchScalarGridSpec(
            num_scalar_prefetch=0,
            grid=(B // nb,),
            in_specs=[
                pl.BlockSpec((HW, nb, C), lambda b: (0, b, 0)),
                pl.BlockSpec((C, I), lambda b: (0, 0)),
                pl.BlockSpec((1, I), lambda b: (0, 0)),
                pl.BlockSpec((I, C), lambda b: (0, 0)),
                pl.BlockSpec((1, C), lambda b: (0, 0)),
            ],
            out_specs=pl.BlockSpec((HW, nb, C), lambda b: (0, b, 0)),
        ),
        compiler_params=pltpu.CompilerParams(
            dimension_semantics=("parallel",),
            vmem_limit_bytes=vmem,
        ),
    )(x_t, wdt, bd2, wut, bu2)

    return out.reshape(H, W, B, C).transpose(2, 3, 0, 1)
